# Initial kernel scaffold; baseline (speedup 1.0000x reference)
import jax
import jax.numpy as jnp
from jax.experimental import pallas as pl


def kernel(indices, table):
    raise NotImplementedError("write your pallas kernel here")


# speedup vs baseline: 3.3681x; 3.3681x over previous
import functools

import jax
import jax.numpy as jnp
from jax import lax
from jax.experimental import pallas as pl
from jax.experimental.pallas import tpu as pltpu
from jax.experimental.pallas import tpu_sc as plsc

NC = 2
NS = 16
NW = NC * NS

IDXW = 128
NSTREAM = 8
CHUNK = IDXW * NSTREAM


def _embed_flat(indices2d, table):
    n_rows, _ = indices2d.shape
    B = n_rows * IDXW
    D = table.shape[1]
    b_per_w = B // NW
    n_chunks = b_per_w // CHUNK
    rows_per_w = b_per_w // IDXW

    mesh = plsc.VectorSubcoreMesh(core_axis_name="c", subcore_axis_name="s")

    @functools.partial(
        pl.kernel,
        mesh=mesh,
        out_type=jax.ShapeDtypeStruct((B, D), jnp.float32),
        scratch_types=[
            pltpu.VMEM((NSTREAM, IDXW), jnp.int32),
            pltpu.VMEM((CHUNK, D), jnp.float32),
            pltpu.SemaphoreType.DMA,
        ],
        compiler_params=pltpu.CompilerParams(use_tc_tiling_on_sc=False),
    )
    def k(idx_hbm, table_hbm, out_hbm, idx_v, rows_v, sem):
        wid = lax.axis_index("s") * NC + lax.axis_index("c")
        row_base = wid * rows_per_w

        def body(c, _):
            r0 = row_base + c * NSTREAM
            pltpu.sync_copy(idx_hbm.at[pl.ds(r0, NSTREAM)], idx_v)
            copies = []
            for j in range(NSTREAM):
                copies.append(
                    pltpu.async_copy(
                        table_hbm.at[idx_v.at[j]],
                        rows_v.at[pl.ds(j * IDXW, IDXW)],
                        sem,
                    )
                )
            for cp in copies:
                cp.wait()
            out0 = r0 * IDXW
            pltpu.sync_copy(rows_v, out_hbm.at[pl.ds(out0, CHUNK)])
            return ()

        lax.fori_loop(0, n_chunks, body, (), unroll=False)

    return k(indices2d, table)


def kernel(indices, table):
    B_, T = indices.shape
    D = table.shape[1]
    flat = indices.reshape(B_ * T // IDXW, IDXW)
    out = _embed_flat(flat, table)
    return out.reshape(B_, T, D)


# speedup vs baseline: 5.2383x; 1.5553x over previous
import functools

import jax
import jax.numpy as jnp
from jax import lax
from jax.experimental import pallas as pl
from jax.experimental.pallas import tpu as pltpu
from jax.experimental.pallas import tpu_sc as plsc

NC = 2
NS = 16
NW = NC * NS

IDXW = 128
NSTREAM = 8
CHUNK = IDXW * NSTREAM


def _embed_flat(indices2d, table):
    n_rows, _ = indices2d.shape
    B = n_rows * IDXW
    D = table.shape[1]
    b_per_w = B // NW
    n_chunks = b_per_w // CHUNK
    rows_per_w = b_per_w // IDXW

    mesh = plsc.VectorSubcoreMesh(core_axis_name="c", subcore_axis_name="s")

    @functools.partial(
        pl.kernel,
        mesh=mesh,
        out_type=jax.ShapeDtypeStruct((B, D), jnp.float32),
        scratch_types=[
            pltpu.VMEM((NSTREAM, IDXW), jnp.int32),
            pltpu.VMEM((CHUNK, D), jnp.float32),
            pltpu.VMEM_SHARED(table.shape, jnp.float32),
            pltpu.SemaphoreType.DMA,
        ],
        compiler_params=pltpu.CompilerParams(use_tc_tiling_on_sc=False),
    )
    def k(idx_hbm, table_hbm, out_hbm, idx_v, rows_v, tab_v, sem):
        sid = lax.axis_index("s")
        wid = sid * NC + lax.axis_index("c")
        row_base = wid * rows_per_w

        @pl.when(sid == 0)
        def _stage():
            pltpu.sync_copy(table_hbm, tab_v)

        plsc.subcore_barrier()

        def body(c, _):
            r0 = row_base + c * NSTREAM
            pltpu.sync_copy(idx_hbm.at[pl.ds(r0, NSTREAM)], idx_v)
            copies = []
            for j in range(NSTREAM):
                copies.append(
                    pltpu.async_copy(
                        tab_v.at[idx_v.at[j]],
                        rows_v.at[pl.ds(j * IDXW, IDXW)],
                        sem,
                    )
                )
            for cp in copies:
                cp.wait()
            out0 = r0 * IDXW
            pltpu.sync_copy(rows_v, out_hbm.at[pl.ds(out0, CHUNK)])
            return ()

        lax.fori_loop(0, n_chunks, body, (), unroll=False)

    return k(indices2d, table)


def kernel(indices, table):
    B_, T = indices.shape
    D = table.shape[1]
    flat = indices.reshape(B_ * T // IDXW, IDXW)
    out = _embed_flat(flat, table)
    return out.reshape(B_, T, D)


# speedup vs baseline: 5.8126x; 1.1096x over previous
import functools

import jax
import jax.numpy as jnp
from jax import lax
from jax.experimental import pallas as pl
from jax.experimental.pallas import tpu as pltpu
from jax.experimental.pallas import tpu_sc as plsc

NC = 2
NS = 16
NW = NC * NS

IDXW = 128
NSTREAM = 5
CHUNK = IDXW * NSTREAM
NBUF = 2


def _embed_flat(indices2d, table):
    n_rows, _ = indices2d.shape
    B = n_rows * IDXW
    D = table.shape[1]
    b_per_w = B // NW
    n_chunks = b_per_w // CHUNK
    assert n_chunks % NBUF == 0 and b_per_w % CHUNK == 0

    mesh = plsc.VectorSubcoreMesh(core_axis_name="c", subcore_axis_name="s")

    @functools.partial(
        pl.kernel,
        mesh=mesh,
        out_type=jax.ShapeDtypeStruct((B, D), jnp.float32),
        scratch_types=[
            [pltpu.VMEM((NSTREAM, IDXW), jnp.int32) for _ in range(NBUF)],
            [pltpu.VMEM((CHUNK, D), jnp.float32) for _ in range(NBUF)],
            pltpu.VMEM_SHARED(table.shape, jnp.float32),
            [pltpu.SemaphoreType.DMA for _ in range(NBUF)],
            [pltpu.SemaphoreType.DMA for _ in range(NBUF)],
            pltpu.SemaphoreType.DMA,
        ],
        compiler_params=pltpu.CompilerParams(use_tc_tiling_on_sc=False),
    )
    def k(idx_hbm, table_hbm, out_hbm, idx_v, rows_v, tab_sh, sem_idx,
          sem_out, sem_g):
        sid = lax.axis_index("s")
        wid = sid * NC + lax.axis_index("c")
        row_base = wid * (b_per_w // IDXW)

        @pl.when(sid == 0)
        def _stage():
            pltpu.sync_copy(table_hbm, tab_sh)

        plsc.subcore_barrier()

        def idx_copy(c, b):
            return pltpu.make_async_copy(
                idx_hbm.at[pl.ds(row_base + c * NSTREAM, NSTREAM)],
                idx_v[b],
                sem_idx[b],
            )

        def out_copy(c, b):
            return pltpu.make_async_copy(
                rows_v[b],
                out_hbm.at[pl.ds((row_base + c * NSTREAM) * IDXW, CHUNK)],
                sem_out[b],
            )

        for b in range(NBUF):
            idx_copy(b, b).start()

        def body(g, _):
            for b in range(NBUF):
                c = g * NBUF + b
                idx_copy(c, b).wait()

                @pl.when(g > 0)
                def _reuse():
                    out_copy(c - NBUF, b).wait()

                copies = []
                for j in range(NSTREAM):
                    copies.append(
                        pltpu.async_copy(
                            tab_sh.at[idx_v[b].at[j]],
                            rows_v[b].at[pl.ds(j * IDXW, IDXW)],
                            sem_g,
                        )
                    )
                for cp in copies:
                    cp.wait()

                out_copy(c, b).start()
                cn = jnp.minimum(c + NBUF, n_chunks - 1)
                idx_copy(cn, b).start()
            return ()

        lax.fori_loop(0, n_chunks // NBUF, body, (), unroll=False)

        for b in range(NBUF):
            idx_copy(0, b).wait()
            out_copy(n_chunks - NBUF + b, b).wait()

    return k(indices2d, table)


def kernel(indices, table):
    B_, T = indices.shape
    D = table.shape[1]
    flat = indices.reshape(B_ * T // IDXW, IDXW)
    out = _embed_flat(flat, table)
    return out.reshape(B_, T, D)


# speedup vs baseline: 5.8172x; 1.0008x over previous
import functools

import jax
import jax.numpy as jnp
from jax import lax
from jax.experimental import pallas as pl
from jax.experimental.pallas import tpu as pltpu
from jax.experimental.pallas import tpu_sc as plsc

NC = 2
NS = 16
NW = NC * NS

NB = 2
IDXW = 40
NBUF = 2


def _embed(indices, table):
    Bb, T = indices.shape
    D = table.shape[1]
    b_per_w = Bb // NW
    n_chunks = b_per_w // NB
    assert b_per_w % NB == 0 and n_chunks % NBUF == 0 and T % IDXW == 0

    mesh = plsc.VectorSubcoreMesh(core_axis_name="c", subcore_axis_name="s")

    @functools.partial(
        pl.kernel,
        mesh=mesh,
        out_type=jax.ShapeDtypeStruct((Bb, T, D), jnp.float32),
        scratch_types=[
            [pltpu.VMEM((NB, T), jnp.int32) for _ in range(NBUF)],
            [pltpu.VMEM((NB, T, D), jnp.float32) for _ in range(NBUF)],
            pltpu.VMEM_SHARED(table.shape, jnp.float32),
            [pltpu.SemaphoreType.DMA for _ in range(NBUF)],
            [pltpu.SemaphoreType.DMA for _ in range(NBUF)],
            pltpu.SemaphoreType.DMA,
        ],
        compiler_params=pltpu.CompilerParams(use_tc_tiling_on_sc=False),
    )
    def k(idx_hbm, table_hbm, out_hbm, idx_v, rows_v, tab_sh, sem_idx,
          sem_out, sem_g):
        sid = lax.axis_index("s")
        wid = sid * NC + lax.axis_index("c")
        b_base = wid * b_per_w

        @pl.when(sid == 0)
        def _stage():
            pltpu.sync_copy(table_hbm, tab_sh)

        plsc.subcore_barrier()

        def idx_copy(c, b):
            return pltpu.make_async_copy(
                idx_hbm.at[pl.ds(b_base + c * NB, NB)],
                idx_v[b],
                sem_idx[b],
            )

        def out_copy(c, b):
            return pltpu.make_async_copy(
                rows_v[b],
                out_hbm.at[pl.ds(b_base + c * NB, NB)],
                sem_out[b],
            )

        for b in range(NBUF):
            idx_copy(b, b).start()

        def body(g, _):
            for b in range(NBUF):
                c = g * NBUF + b
                idx_copy(c, b).wait()

                @pl.when(g > 0)
                def _reuse():
                    out_copy(c - NBUF, b).wait()

                copies = []
                for nb in range(NB):
                    for j in range(T // IDXW):
                        copies.append(
                            pltpu.async_copy(
                                tab_sh.at[idx_v[b].at[nb, pl.ds(j * IDXW, IDXW)]],
                                rows_v[b].at[nb, pl.ds(j * IDXW, IDXW)],
                                sem_g,
                            )
                        )
                for cp in copies:
                    cp.wait()

                out_copy(c, b).start()
                cn = jnp.minimum(c + NBUF, n_chunks - 1)
                idx_copy(cn, b).start()
            return ()

        lax.fori_loop(0, n_chunks // NBUF, body, (), unroll=False)

        for b in range(NBUF):
            idx_copy(0, b).wait()
            out_copy(n_chunks - NBUF + b, b).wait()

    return k(indices, table)


def kernel(indices, table):
    return _embed(indices, table)
